# baseline (speedup 1.0000x reference)
import functools

import jax
import jax.numpy as jnp
from jax import lax
from jax.experimental import pallas as pl
from jax.experimental.pallas import tpu as pltpu
from jax.experimental.pallas import tpu_sc as plsc

N = 10000
E = 320000
K = 3
EPS = 1e-15
CIN = 128
COUT = 128
KC = K * COUT

NC = 2
NS = 16
NW = NC * NS
E_TC = 81920
E_SC = E - E_TC
E_PER = E_SC // NW
CHUNK = 40
NCHUNK = E_PER // CHUNK
E_PER_C = E // NW
NCHUNK_C = E_PER_C // CHUNK
NPAD = 10240
RPT = NPAD // NS
ZROWS = 128
GE = 8
NGRP = E_TC // GE
NTILE = NGRP // 128

_ROW_BLK = 2000


def _pre_body(x_ref, g_ref, b_ref, xg_ref, rt_ref):
    xG = jnp.dot(x_ref[...], g_ref[...], preferred_element_type=jnp.float32)
    xg_ref[...] = xG[:, :KC]
    rt_ref[...] = xG[:, KC:] + b_ref[...]


def _pre(x, G, b):
    return pl.pallas_call(
        _pre_body,
        grid=(N // _ROW_BLK,),
        in_specs=[
            pl.BlockSpec((_ROW_BLK, CIN), lambda i: (i, 0)),
            pl.BlockSpec((CIN, KC + COUT), lambda i: (0, 0)),
            pl.BlockSpec((1, COUT), lambda i: (0, 0)),
        ],
        out_specs=[
            pl.BlockSpec((_ROW_BLK, KC), lambda i: (i, 0)),
            pl.BlockSpec((_ROW_BLK, COUT), lambda i: (i, 0)),
        ],
        out_shape=[
            jax.ShapeDtypeStruct((N, KC), jnp.float32),
            jax.ShapeDtypeStruct((N, COUT), jnp.float32),
        ],
    )(x, G, b)


def _w_body(a_ref, mu_ref, iv_ref, w_ref):
    a = a_ref[...]
    for i in range(4 * K):
        d = a - mu_ref[i]
        w_ref[i] = jnp.exp((-0.5) * iv_ref[i] * d * d)


def _w_tc(a2, mu_flat, iv_flat):
    rows = E // 128
    return pl.pallas_call(
        _w_body,
        in_specs=[
            pl.BlockSpec((rows, 128), lambda: (0, 0)),
            pl.BlockSpec(memory_space=pltpu.SMEM),
            pl.BlockSpec(memory_space=pltpu.SMEM),
        ],
        out_shape=jax.ShapeDtypeStruct((4 * K, rows, 128), jnp.float32),
    )(a2, mu_flat, iv_flat)


def _inv_body(p_ref, inv_ref):
    s = p_ref[0, :, 0:1] + p_ref[1, :, 0:1]
    inv_ref[...] = 1.0 / jnp.clip(s, 1.0, None)


def _inv_tc(cntp):
    return pl.pallas_call(
        _inv_body,
        grid=(N // _ROW_BLK,),
        in_specs=[pl.BlockSpec((NC, _ROW_BLK, COUT), lambda i: (0, i, 0))],
        out_specs=pl.BlockSpec((_ROW_BLK, 1), lambda i: (i, 0)),
        out_shape=jax.ShapeDtypeStruct((N, 1), jnp.float32),
    )(cntp)


def _post_body(has_skip, apply_gelu, *refs):
    if has_skip:
        agg_ref, tcp_ref, inv_ref, rt_ref, skip_ref, out_ref = refs
    else:
        agg_ref, tcp_ref, inv_ref, rt_ref, out_ref = refs
    v = (agg_ref[0] + agg_ref[1] + tcp_ref[...]) * inv_ref[...] + rt_ref[...]
    if has_skip:
        v = v + skip_ref[...]
    if apply_gelu:
        v = 0.5 * v * (1.0 + lax.erf(v * (2.0 ** -0.5)))
    out_ref[...] = v


def _post(aggp, tcp, inv, rt, skip, apply_gelu):
    has_skip = skip is not None
    in_specs = [
        pl.BlockSpec((NC, _ROW_BLK, COUT), lambda i: (0, i, 0)),
        pl.BlockSpec((_ROW_BLK, COUT), lambda i: (i, 0)),
        pl.BlockSpec((_ROW_BLK, 1), lambda i: (i, 0)),
        pl.BlockSpec((_ROW_BLK, COUT), lambda i: (i, 0)),
    ]
    args = [aggp, tcp, inv, rt]
    if has_skip:
        in_specs.append(pl.BlockSpec((_ROW_BLK, COUT), lambda i: (i, 0)))
        args.append(skip)
    return pl.pallas_call(
        functools.partial(_post_body, has_skip, apply_gelu),
        grid=(N // _ROW_BLK,),
        in_specs=in_specs,
        out_specs=pl.BlockSpec((_ROW_BLK, COUT), lambda i: (i, 0)),
        out_shape=jax.ShapeDtypeStruct((N, COUT), jnp.float32),
    )(*args)


def _tc_edge_body(src_ref, dst_ref, w0_ref, w1_ref, w2_ref, xg_ref, out_ref,
                  tmp_ref):
    out_ref[...] = jnp.zeros((N, COUT), jnp.float32)

    def group(gb, _):
        base = gb * GE
        for i in range(GE):
            s = src_ref[base + i]
            tmp_ref[i, :] = xg_ref[s, :]
        rows = tmp_ref[...]
        t = gb // 128
        l = gb - t * 128
        oh = (lax.broadcasted_iota(jnp.int32, (1, 128), 1) == l
              ).astype(jnp.float32)
        w0 = jnp.sum(w0_ref[t] * oh, axis=1, keepdims=True)
        w1 = jnp.sum(w1_ref[t] * oh, axis=1, keepdims=True)
        w2 = jnp.sum(w2_ref[t] * oh, axis=1, keepdims=True)
        msg = (rows[:, :COUT] * w0 + rows[:, COUT:2 * COUT] * w1
               + rows[:, 2 * COUT:] * w2)
        for i in range(GE):
            d = dst_ref[base + i]
            out_ref[d, :] = out_ref[d, :] + msg[i, :]
        return 0

    lax.fori_loop(0, E_TC // GE, group, 0)


def _tc_edge(xg, src_tc, dst_tc, w0, w1, w2):
    return pl.pallas_call(
        _tc_edge_body,
        in_specs=[
            pl.BlockSpec(memory_space=pltpu.SMEM),
            pl.BlockSpec(memory_space=pltpu.SMEM),
            pl.BlockSpec((NTILE, GE, 128), lambda: (0, 0, 0)),
            pl.BlockSpec((NTILE, GE, 128), lambda: (0, 0, 0)),
            pl.BlockSpec((NTILE, GE, 128), lambda: (0, 0, 0)),
            pl.BlockSpec((N, KC), lambda: (0, 0)),
        ],
        out_specs=pl.BlockSpec((N, COUT), lambda: (0, 0)),
        out_shape=jax.ShapeDtypeStruct((N, COUT), jnp.float32),
        scratch_shapes=[pltpu.VMEM((GE, KC), jnp.float32)],
    )(src_tc, dst_tc, w0, w1, w2, xg)


def _sc_cnt(dst):
    mesh = plsc.VectorSubcoreMesh(core_axis_name="c", subcore_axis_name="s")

    @functools.partial(
        pl.kernel,
        out_type=jax.ShapeDtypeStruct((NC, NPAD, COUT), jnp.float32),
        mesh=mesh,
        scratch_types=[
            pltpu.VMEM((CHUNK,), jnp.int32),
            pltpu.VMEM((CHUNK, COUT), jnp.float32),
            pltpu.VMEM_SHARED((NPAD, COUT), jnp.float32),
        ],
    )
    def body(dst_hbm, out_hbm, dstv, onesv, acc):
        cid = lax.axis_index("c")
        sid = lax.axis_index("s")
        wid = sid * NC + cid
        zero = jnp.zeros((16,), jnp.float32)
        one = jnp.ones((16,), jnp.float32)

        def z(i, _):
            for q in range(COUT // 16):
                onesv[i, pl.ds(q * 16, 16)] = zero
            return 0

        lax.fori_loop(0, CHUNK, z, 0)
        for cpy in range(RPT // CHUNK):
            pltpu.sync_copy(
                onesv, acc.at[pl.ds(sid * RPT + cpy * CHUNK, CHUNK)])

        def o(i, _):
            onesv[i, pl.ds(0, 16)] = one
            return 0

        lax.fori_loop(0, CHUNK, o, 0)
        plsc.subcore_barrier()

        def chunk(i, _):
            base = wid * E_PER_C + i * CHUNK
            pltpu.sync_copy(dst_hbm.at[pl.ds(base, CHUNK)], dstv)
            pltpu.sync_copy(onesv, acc.at[dstv], add=True)
            return 0

        lax.fori_loop(0, NCHUNK_C, chunk, 0)

        plsc.subcore_barrier()
        pltpu.sync_copy(
            acc.at[pl.ds(sid * RPT, RPT)],
            out_hbm.at[cid, pl.ds(sid * RPT, RPT)],
        )

    return body(dst)


def _sc_edge(xg, src, dst, w0a, w1a, w2a):
    mesh = plsc.VectorSubcoreMesh(core_axis_name="c", subcore_axis_name="s")

    @functools.partial(
        pl.kernel,
        out_type=jax.ShapeDtypeStruct((NC, NPAD, COUT), jnp.float32),
        mesh=mesh,
        scratch_types=[
            pltpu.VMEM((CHUNK,), jnp.int32),
            pltpu.VMEM((CHUNK,), jnp.int32),
            pltpu.VMEM((CHUNK,), jnp.int32),
            pltpu.VMEM((CHUNK,), jnp.int32),
            pltpu.VMEM((48,), jnp.float32),
            pltpu.VMEM((48,), jnp.float32),
            pltpu.VMEM((48,), jnp.float32),
            pltpu.VMEM((48,), jnp.float32),
            pltpu.VMEM((48,), jnp.float32),
            pltpu.VMEM((48,), jnp.float32),
            pltpu.VMEM((CHUNK, KC), jnp.float32),
            pltpu.VMEM((CHUNK, KC), jnp.float32),
            pltpu.VMEM((CHUNK, COUT), jnp.float32),
            pltpu.VMEM((CHUNK, COUT), jnp.float32),
            pltpu.VMEM((48,), jnp.int32),
            pltpu.VMEM((48,), jnp.int32),
            pltpu.SemaphoreType.DMA,
            pltpu.SemaphoreType.DMA,
            pltpu.SemaphoreType.DMA,
            pltpu.SemaphoreType.DMA,
            pltpu.SemaphoreType.DMA,
            pltpu.SemaphoreType.DMA,
            pltpu.VMEM_SHARED((NPAD, COUT), jnp.float32),
        ],
    )
    def body(xg_hbm, src_hbm, dst_hbm, w0_hbm, w1_hbm, w2_hbm, out_hbm,
             idxA, idxB, dstA, dstB, w0A, w0B, w1A, w1B, w2A, w2B,
             rowsA, rowsB, msgA, msgB, dsA, dsB,
             semA0, semA1, semG0, semG1, semS0, semS1, acc):
        cid = lax.axis_index("c")
        sid = lax.axis_index("s")
        wid = sid * NC + cid
        zero = jnp.zeros((16,), jnp.float32)

        slots = [
            (idxA, dstA, w0A, w1A, w2A, rowsA, semA0, semG0,
             msgA, dsA, semS0),
            (idxB, dstB, w0B, w1B, w2B, rowsB, semA1, semG1,
             msgB, dsB, semS1),
        ]

        def lin_descs(i, s):
            base = E_TC + wid * E_PER + i * CHUNK
            return [
                (src_hbm.at[pl.ds(base, CHUNK)], s[0]),
                (dst_hbm.at[pl.ds(base, CHUNK)], s[1]),
                (w0_hbm.at[pl.ds(base, CHUNK)], s[2].at[pl.ds(0, CHUNK)]),
                (w1_hbm.at[pl.ds(base, CHUNK)], s[3].at[pl.ds(0, CHUNK)]),
                (w2_hbm.at[pl.ds(base, CHUNK)], s[4].at[pl.ds(0, CHUNK)]),
            ]

        def issue_lin(i, b):
            s = slots[b]
            for a, d in lin_descs(i, s):
                pltpu.async_copy(a, d, s[6])

        def wait_lin(i, b):
            s = slots[b]
            for a, d in lin_descs(i, s):
                pltpu.make_async_copy(a, d, s[6]).wait()

        def issue_gather(b):
            s = slots[b]
            pltpu.async_copy(xg_hbm.at[s[0]], s[5], s[7])

        def wait_gather(b):
            s = slots[b]
            pltpu.make_async_copy(xg_hbm.at[s[0]], s[5], s[7]).wait()

        def issue_scatter(b):
            s = slots[b]
            pltpu.async_copy(
                s[8], acc.at[s[9].at[pl.ds(0, CHUNK)]], s[10], add=True)

        def wait_scatter(b):
            s = slots[b]
            pltpu.make_async_copy(
                s[8], acc.at[s[9].at[pl.ds(0, CHUNK)]], s[10]).wait()

        def compute(b):
            s = slots[b]
            rows = s[5]
            msg = s[8]

            def do_group(g0, gs, jbase):
                wv0 = s[2][pl.ds(g0, 16)]
                wv1 = s[3][pl.ds(g0, 16)]
                wv2 = s[4][pl.ds(g0, 16)]
                for t in range(gs):
                    j = jbase + t
                    w0 = wv0[t]
                    w1 = wv1[t]
                    w2 = wv2[t]
                    for q in range(COUT // 16):
                        r0 = rows[j, pl.ds(q * 16, 16)]
                        r1 = rows[j, pl.ds(COUT + q * 16, 16)]
                        r2 = rows[j, pl.ds(2 * COUT + q * 16, 16)]
                        msg[j, pl.ds(q * 16, 16)] = (
                            r0 * w0 + r1 * w1 + r2 * w2)

            def grp(g2, _):
                do_group(g2 * 16, 16, g2 * 16)
                return 0

            lax.fori_loop(0, 2, grp, 0)
            do_group(32, CHUNK - 32, 32)

        def z(i, _):
            for q in range(COUT // 16):
                msgA[i, pl.ds(q * 16, 16)] = zero
            return 0

        lax.fori_loop(0, CHUNK, z, 0)
        for cpy in range(RPT // CHUNK):
            pltpu.sync_copy(
                msgA, acc.at[pl.ds(sid * RPT + cpy * CHUNK, CHUNK)])
        plsc.subcore_barrier()

        issue_lin(0, 0)
        wait_lin(0, 0)
        issue_gather(0)
        issue_lin(1, 1)

        def pair(g, _):
            for b in (0, 1):
                i = 2 * g + b
                s = slots[b]
                wait_gather(b)

                @pl.when(i + 1 < NCHUNK)
                def _():
                    wait_lin(i + 1, 1 - b)
                    issue_gather(1 - b)

                @pl.when(i >= 2)
                def _():
                    wait_scatter(b)

                for off in (0, 16, 24):
                    s[9][pl.ds(off, 16)] = s[1][pl.ds(off, 16)]
                compute(b)
                issue_scatter(b)

                @pl.when(i + 2 < NCHUNK)
                def _():
                    issue_lin(i + 2, b)
            return 0

        lax.fori_loop(0, NCHUNK // 2, pair, 0)
        wait_scatter(0)
        wait_scatter(1)

        plsc.subcore_barrier()
        pltpu.sync_copy(
            acc.at[pl.ds(sid * RPT, RPT)],
            out_hbm.at[cid, pl.ds(sid * RPT, RPT)],
        )

    return body(xg, src, dst, w0a, w1a, w2a)


def kernel(x, edge_index, edge_attr, params):
    src = edge_index[0]
    dst = edge_index[1]

    G = [jnp.concatenate([p['g'], p['root']], axis=1) for p in params]
    biases = [p['bias'].reshape(1, COUT) for p in params]
    mu_flat = jnp.concatenate([p['mu'][:, 0] for p in params])
    sig_flat = jnp.concatenate([p['sigma'][:, 0] for p in params])
    iv_flat = 1.0 / (EPS + sig_flat * sig_flat)

    a2 = edge_attr.reshape(E // 128, 128)
    w_all = _w_tc(a2, mu_flat, iv_flat).reshape(4, K, E)

    cntp = _sc_cnt(dst)
    inv = _inv_tc(cntp)

    src_tc = src[:E_TC]
    dst_tc = dst[:E_TC]

    h = x
    saved = None
    for li in range(4):
        xg, rt = _pre(h, G[li], biases[li])
        aggp = _sc_edge(xg, src, dst,
                        w_all[li, 0], w_all[li, 1], w_all[li, 2])
        tcp = _tc_edge(xg, src_tc, dst_tc,
                       w_all[li, 0, :E_TC].reshape(NTILE, 128, GE)
                       .transpose(0, 2, 1),
                       w_all[li, 1, :E_TC].reshape(NTILE, 128, GE)
                       .transpose(0, 2, 1),
                       w_all[li, 2, :E_TC].reshape(NTILE, 128, GE)
                       .transpose(0, 2, 1))
        skip = saved if li == 2 else None
        h = _post(aggp, tcp, inv, rt, skip, apply_gelu=(li < 3))
        if li == 0:
            saved = h
    return h


# speedup vs baseline: 1.3077x; 1.3077x over previous
import functools

import jax
import jax.numpy as jnp
from jax import lax
from jax.experimental import pallas as pl
from jax.experimental.pallas import tpu as pltpu
from jax.experimental.pallas import tpu_sc as plsc

N = 10000
E = 320000
K = 3
EPS = 1e-15
CIN = 128
COUT = 128
KC = K * COUT

NC = 2
NS = 16
NW = NC * NS
E_TC = 61440
E_SC = E - E_TC
E_PER = E_SC // NW
CHUNK = 40
NCHUNK = E_PER // CHUNK
E_PER_C = E // NW
NCHUNK_C = E_PER_C // CHUNK
NPAD = 10240
RPT = NPAD // NS
ZROWS = 128
GE = 8
NGRP = E_TC // GE
NTILE = NGRP // 128

_ROW_BLK = 2000


def _pre_body(x_ref, g_ref, b_ref, xg_ref, rt_ref):
    xG = jnp.dot(x_ref[...], g_ref[...], preferred_element_type=jnp.float32)
    xg_ref[...] = xG[:, :KC]
    rt_ref[...] = xG[:, KC:] + b_ref[...]


def _pre(x, G, b):
    return pl.pallas_call(
        _pre_body,
        grid=(N // _ROW_BLK,),
        in_specs=[
            pl.BlockSpec((_ROW_BLK, CIN), lambda i: (i, 0)),
            pl.BlockSpec((CIN, KC + COUT), lambda i: (0, 0)),
            pl.BlockSpec((1, COUT), lambda i: (0, 0)),
        ],
        out_specs=[
            pl.BlockSpec((_ROW_BLK, KC), lambda i: (i, 0)),
            pl.BlockSpec((_ROW_BLK, COUT), lambda i: (i, 0)),
        ],
        out_shape=[
            jax.ShapeDtypeStruct((N, KC), jnp.float32),
            jax.ShapeDtypeStruct((N, COUT), jnp.float32),
        ],
    )(x, G, b)


def _w_body(a_ref, mu_ref, iv_ref, w_ref):
    a = a_ref[...]
    for i in range(4 * K):
        d = a - mu_ref[i]
        w_ref[i] = jnp.exp((-0.5) * iv_ref[i] * d * d)


def _w_tc(a2, mu_flat, iv_flat):
    rows = E // 128
    return pl.pallas_call(
        _w_body,
        in_specs=[
            pl.BlockSpec((rows, 128), lambda: (0, 0)),
            pl.BlockSpec(memory_space=pltpu.SMEM),
            pl.BlockSpec(memory_space=pltpu.SMEM),
        ],
        out_shape=jax.ShapeDtypeStruct((4 * K, rows, 128), jnp.float32),
    )(a2, mu_flat, iv_flat)


def _inv_body(p_ref, inv_ref):
    s = p_ref[0, :, 0:1] + p_ref[1, :, 0:1]
    inv_ref[...] = 1.0 / jnp.clip(s, 1.0, None)


def _inv_tc(cntp):
    return pl.pallas_call(
        _inv_body,
        grid=(N // _ROW_BLK,),
        in_specs=[pl.BlockSpec((NC, _ROW_BLK, COUT), lambda i: (0, i, 0))],
        out_specs=pl.BlockSpec((_ROW_BLK, 1), lambda i: (i, 0)),
        out_shape=jax.ShapeDtypeStruct((N, 1), jnp.float32),
    )(cntp)


def _post_body(has_skip, apply_gelu, *refs):
    if has_skip:
        agg_ref, tcp_ref, inv_ref, rt_ref, skip_ref, out_ref = refs
    else:
        agg_ref, tcp_ref, inv_ref, rt_ref, out_ref = refs
    v = (agg_ref[0] + agg_ref[1] + tcp_ref[...]) * inv_ref[...] + rt_ref[...]
    if has_skip:
        v = v + skip_ref[...]
    if apply_gelu:
        v = 0.5 * v * (1.0 + lax.erf(v * (2.0 ** -0.5)))
    out_ref[...] = v


def _post(aggp, tcp, inv, rt, skip, apply_gelu):
    has_skip = skip is not None
    in_specs = [
        pl.BlockSpec((NC, _ROW_BLK, COUT), lambda i: (0, i, 0)),
        pl.BlockSpec((_ROW_BLK, COUT), lambda i: (i, 0)),
        pl.BlockSpec((_ROW_BLK, 1), lambda i: (i, 0)),
        pl.BlockSpec((_ROW_BLK, COUT), lambda i: (i, 0)),
    ]
    args = [aggp, tcp, inv, rt]
    if has_skip:
        in_specs.append(pl.BlockSpec((_ROW_BLK, COUT), lambda i: (i, 0)))
        args.append(skip)
    return pl.pallas_call(
        functools.partial(_post_body, has_skip, apply_gelu),
        grid=(N // _ROW_BLK,),
        in_specs=in_specs,
        out_specs=pl.BlockSpec((_ROW_BLK, COUT), lambda i: (i, 0)),
        out_shape=jax.ShapeDtypeStruct((N, COUT), jnp.float32),
    )(*args)


def _tc_edge_body(src_ref, dst_ref, w0_ref, w1_ref, w2_ref, xg_ref, out_ref,
                  tmp_ref):
    out_ref[...] = jnp.zeros((N, COUT), jnp.float32)

    def group(gb, _):
        base = gb * GE
        for i in range(GE):
            s = src_ref[base + i]
            tmp_ref[i, :] = xg_ref[s, :]
        rows = tmp_ref[...]
        t = gb // 128
        l = gb - t * 128
        oh = (lax.broadcasted_iota(jnp.int32, (1, 128), 1) == l
              ).astype(jnp.float32)
        w0 = jnp.sum(w0_ref[t] * oh, axis=1, keepdims=True)
        w1 = jnp.sum(w1_ref[t] * oh, axis=1, keepdims=True)
        w2 = jnp.sum(w2_ref[t] * oh, axis=1, keepdims=True)
        msg = (rows[:, :COUT] * w0 + rows[:, COUT:2 * COUT] * w1
               + rows[:, 2 * COUT:] * w2)
        for i in range(GE):
            d = dst_ref[base + i]
            out_ref[d, :] = out_ref[d, :] + msg[i, :]
        return 0

    lax.fori_loop(0, E_TC // GE, group, 0)


def _tc_edge(xg, src_tc, dst_tc, w0, w1, w2):
    return pl.pallas_call(
        _tc_edge_body,
        in_specs=[
            pl.BlockSpec(memory_space=pltpu.SMEM),
            pl.BlockSpec(memory_space=pltpu.SMEM),
            pl.BlockSpec((NTILE, GE, 128), lambda: (0, 0, 0)),
            pl.BlockSpec((NTILE, GE, 128), lambda: (0, 0, 0)),
            pl.BlockSpec((NTILE, GE, 128), lambda: (0, 0, 0)),
            pl.BlockSpec((N, KC), lambda: (0, 0)),
        ],
        out_specs=pl.BlockSpec((N, COUT), lambda: (0, 0)),
        out_shape=jax.ShapeDtypeStruct((N, COUT), jnp.float32),
        scratch_shapes=[pltpu.VMEM((GE, KC), jnp.float32)],
    )(src_tc, dst_tc, w0, w1, w2, xg)


def _sc_cnt(dst):
    mesh = plsc.VectorSubcoreMesh(core_axis_name="c", subcore_axis_name="s")

    @functools.partial(
        pl.kernel,
        out_type=jax.ShapeDtypeStruct((NC, NPAD, COUT), jnp.float32),
        mesh=mesh,
        scratch_types=[
            pltpu.VMEM((CHUNK,), jnp.int32),
            pltpu.VMEM((CHUNK, COUT), jnp.float32),
            pltpu.VMEM_SHARED((NPAD, COUT), jnp.float32),
        ],
    )
    def body(dst_hbm, out_hbm, dstv, onesv, acc):
        cid = lax.axis_index("c")
        sid = lax.axis_index("s")
        wid = sid * NC + cid
        zero = jnp.zeros((16,), jnp.float32)
        one = jnp.ones((16,), jnp.float32)

        def z(i, _):
            for q in range(COUT // 16):
                onesv[i, pl.ds(q * 16, 16)] = zero
            return 0

        lax.fori_loop(0, CHUNK, z, 0)
        for cpy in range(RPT // CHUNK):
            pltpu.sync_copy(
                onesv, acc.at[pl.ds(sid * RPT + cpy * CHUNK, CHUNK)])

        def o(i, _):
            onesv[i, pl.ds(0, 16)] = one
            return 0

        lax.fori_loop(0, CHUNK, o, 0)
        plsc.subcore_barrier()

        def chunk(i, _):
            base = wid * E_PER_C + i * CHUNK
            pltpu.sync_copy(dst_hbm.at[pl.ds(base, CHUNK)], dstv)
            pltpu.sync_copy(onesv, acc.at[dstv], add=True)
            return 0

        lax.fori_loop(0, NCHUNK_C, chunk, 0)

        plsc.subcore_barrier()
        pltpu.sync_copy(
            acc.at[pl.ds(sid * RPT, RPT)],
            out_hbm.at[cid, pl.ds(sid * RPT, RPT)],
        )

    return body(dst)


def _sc_edge(xg, src, dst, w0a, w1a, w2a):
    mesh = plsc.VectorSubcoreMesh(core_axis_name="c", subcore_axis_name="s")

    @functools.partial(
        pl.kernel,
        out_type=jax.ShapeDtypeStruct((NC, NPAD, COUT), jnp.float32),
        mesh=mesh,
        scratch_types=[
            pltpu.VMEM((CHUNK,), jnp.int32),
            pltpu.VMEM((CHUNK,), jnp.int32),
            pltpu.VMEM((CHUNK,), jnp.int32),
            pltpu.VMEM((CHUNK,), jnp.int32),
            pltpu.VMEM((48,), jnp.float32),
            pltpu.VMEM((48,), jnp.float32),
            pltpu.VMEM((48,), jnp.float32),
            pltpu.VMEM((48,), jnp.float32),
            pltpu.VMEM((48,), jnp.float32),
            pltpu.VMEM((48,), jnp.float32),
            pltpu.VMEM((CHUNK, KC), jnp.float32),
            pltpu.VMEM((CHUNK, KC), jnp.float32),
            pltpu.VMEM((CHUNK, COUT), jnp.float32),
            pltpu.VMEM((CHUNK, COUT), jnp.float32),
            pltpu.VMEM((48,), jnp.int32),
            pltpu.VMEM((48,), jnp.int32),
            pltpu.SemaphoreType.DMA,
            pltpu.SemaphoreType.DMA,
            pltpu.SemaphoreType.DMA,
            pltpu.SemaphoreType.DMA,
            pltpu.SemaphoreType.DMA,
            pltpu.SemaphoreType.DMA,
            pltpu.VMEM_SHARED((NPAD, COUT), jnp.float32),
        ],
    )
    def body(xg_hbm, src_hbm, dst_hbm, w0_hbm, w1_hbm, w2_hbm, out_hbm,
             idxA, idxB, dstA, dstB, w0A, w0B, w1A, w1B, w2A, w2B,
             rowsA, rowsB, msgA, msgB, dsA, dsB,
             semA0, semA1, semG0, semG1, semS0, semS1, acc):
        cid = lax.axis_index("c")
        sid = lax.axis_index("s")
        wid = sid * NC + cid
        zero = jnp.zeros((16,), jnp.float32)

        slots = [
            (idxA, dstA, w0A, w1A, w2A, rowsA, semA0, semG0,
             msgA, dsA, semS0),
            (idxB, dstB, w0B, w1B, w2B, rowsB, semA1, semG1,
             msgB, dsB, semS1),
        ]

        def lin_descs(i, s):
            base = E_TC + wid * E_PER + i * CHUNK
            return [
                (src_hbm.at[pl.ds(base, CHUNK)], s[0]),
                (dst_hbm.at[pl.ds(base, CHUNK)], s[1]),
                (w0_hbm.at[pl.ds(base, CHUNK)], s[2].at[pl.ds(0, CHUNK)]),
                (w1_hbm.at[pl.ds(base, CHUNK)], s[3].at[pl.ds(0, CHUNK)]),
                (w2_hbm.at[pl.ds(base, CHUNK)], s[4].at[pl.ds(0, CHUNK)]),
            ]

        def issue_lin(i, b):
            s = slots[b]
            for a, d in lin_descs(i, s):
                pltpu.async_copy(a, d, s[6])

        def wait_lin(i, b):
            s = slots[b]
            for a, d in lin_descs(i, s):
                pltpu.make_async_copy(a, d, s[6]).wait()

        def issue_gather(b):
            s = slots[b]
            pltpu.async_copy(xg_hbm.at[s[0]], s[5], s[7])

        def wait_gather(b):
            s = slots[b]
            pltpu.make_async_copy(xg_hbm.at[s[0]], s[5], s[7]).wait()

        def issue_scatter(b):
            s = slots[b]
            pltpu.async_copy(
                s[8], acc.at[s[9].at[pl.ds(0, CHUNK)]], s[10], add=True)

        def wait_scatter(b):
            s = slots[b]
            pltpu.make_async_copy(
                s[8], acc.at[s[9].at[pl.ds(0, CHUNK)]], s[10]).wait()

        def compute(b):
            s = slots[b]
            rows = s[5]
            msg = s[8]

            def do_group(g0, gs, jbase):
                wv0 = s[2][pl.ds(g0, 16)]
                wv1 = s[3][pl.ds(g0, 16)]
                wv2 = s[4][pl.ds(g0, 16)]
                for t in range(gs):
                    j = jbase + t
                    w0 = wv0[t]
                    w1 = wv1[t]
                    w2 = wv2[t]
                    for q in range(COUT // 16):
                        r0 = rows[j, pl.ds(q * 16, 16)]
                        r1 = rows[j, pl.ds(COUT + q * 16, 16)]
                        r2 = rows[j, pl.ds(2 * COUT + q * 16, 16)]
                        msg[j, pl.ds(q * 16, 16)] = (
                            r0 * w0 + r1 * w1 + r2 * w2)

            def grp(g2, _):
                do_group(g2 * 16, 16, g2 * 16)
                return 0

            lax.fori_loop(0, 2, grp, 0)
            do_group(32, CHUNK - 32, 32)

        def z(i, _):
            for q in range(COUT // 16):
                msgA[i, pl.ds(q * 16, 16)] = zero
            return 0

        lax.fori_loop(0, CHUNK, z, 0)
        for cpy in range(RPT // CHUNK):
            pltpu.sync_copy(
                msgA, acc.at[pl.ds(sid * RPT + cpy * CHUNK, CHUNK)])
        plsc.subcore_barrier()

        issue_lin(0, 0)
        wait_lin(0, 0)
        issue_gather(0)
        issue_lin(1, 1)

        def pair(g, _):
            for b in (0, 1):
                i = 2 * g + b
                s = slots[b]
                wait_gather(b)

                @pl.when(i + 1 < NCHUNK)
                def _():
                    wait_lin(i + 1, 1 - b)
                    issue_gather(1 - b)

                @pl.when(i >= 2)
                def _():
                    wait_scatter(b)

                for off in (0, 16, 24):
                    s[9][pl.ds(off, 16)] = s[1][pl.ds(off, 16)]
                compute(b)
                issue_scatter(b)

                @pl.when(i + 2 < NCHUNK)
                def _():
                    issue_lin(i + 2, b)
            return 0

        lax.fori_loop(0, NCHUNK // 2, pair, 0)
        wait_scatter(0)
        wait_scatter(1)

        plsc.subcore_barrier()
        pltpu.sync_copy(
            acc.at[pl.ds(sid * RPT, RPT)],
            out_hbm.at[cid, pl.ds(sid * RPT, RPT)],
        )

    return body(xg, src, dst, w0a, w1a, w2a)


def kernel(x, edge_index, edge_attr, params):
    src = edge_index[0]
    dst = edge_index[1]

    G = [jnp.concatenate([p['g'], p['root']], axis=1) for p in params]
    biases = [p['bias'].reshape(1, COUT) for p in params]
    mu_flat = jnp.concatenate([p['mu'][:, 0] for p in params])
    sig_flat = jnp.concatenate([p['sigma'][:, 0] for p in params])
    iv_flat = 1.0 / (EPS + sig_flat * sig_flat)

    a2 = edge_attr.reshape(E // 128, 128)
    w_all = _w_tc(a2, mu_flat, iv_flat).reshape(4, K, E)

    cntp = _sc_cnt(dst)
    inv = _inv_tc(cntp)

    src_tc = src[:E_TC]
    dst_tc = dst[:E_TC]

    h = x
    saved = None
    for li in range(4):
        xg, rt = _pre(h, G[li], biases[li])
        aggp = _sc_edge(xg, src, dst,
                        w_all[li, 0], w_all[li, 1], w_all[li, 2])
        tcp = _tc_edge(xg, src_tc, dst_tc,
                       w_all[li, 0, :E_TC].reshape(NTILE, 128, GE)
                       .transpose(0, 2, 1),
                       w_all[li, 1, :E_TC].reshape(NTILE, 128, GE)
                       .transpose(0, 2, 1),
                       w_all[li, 2, :E_TC].reshape(NTILE, 128, GE)
                       .transpose(0, 2, 1))
        skip = saved if li == 2 else None
        h = _post(aggp, tcp, inv, rt, skip, apply_gelu=(li < 3))
        if li == 0:
            saved = h
    return h
